# baseline (speedup 1.0000x reference)
import functools

import jax
import jax.numpy as jnp
from jax import lax
from jax.experimental import pallas as pl
from jax.experimental.pallas import tpu as pltpu

NBITS = 4
NCODE = 16
CHUNK = 32768


def _hist_body(inv_total, nsteps, xp_hbm, xt_hbm, o_ref, bufp, buft, sems):
    def copies(g, slot):
        src = (xp_hbm, xt_hbm)
        dst = (bufp, buft)
        return [
            pltpu.make_async_copy(src[a].at[n, pl.ds(g, 1), :, :],
                                  dst[a].at[slot, n], sems.at[a, slot])
            for a in range(2)
            for n in range(NBITS)
        ]

    def encode(buf, slot):
        acc = None
        for n in range(NBITS):
            bit = buf[slot, n, 0]
            acc = bit if acc is None else acc * 2.0 + bit
        return acc.astype(jnp.int32)

    io = lax.broadcasted_iota(jnp.int32, (NCODE, 1), 0)
    acc = jnp.zeros((NCODE, NCODE), jnp.float32)
    nbuf = 4
    for h in range(min(nbuf, nsteps)):
        for i, c in enumerate(copies(h, h % nbuf)):
            c.start(priority=i % 2)
    for g in range(nsteps):
        slot = g % nbuf
        for c in copies(g, slot):
            c.wait()
        mp = (encode(bufp, slot) == io).astype(jnp.float32)
        mt = (encode(buft, slot) == io).astype(jnp.float32)
        acc = acc + lax.dot_general(mp, mt, (((1,), (1,)), ((), ())),
                                    preferred_element_type=jnp.float32)
        if g + nbuf < nsteps:
            for i, c in enumerate(copies(g + nbuf, slot)):
                c.start(priority=i % 2)
    o_ref[...] = acc * inv_total


@jax.jit
def kernel(y_pred, y_true):
    b = y_pred.shape[0]
    nsteps = b // CHUNK

    hist_call = pl.pallas_call(
        functools.partial(_hist_body, 1.0 / b, nsteps),
        in_specs=[
            pl.BlockSpec(memory_space=pltpu.MemorySpace.HBM),
            pl.BlockSpec(memory_space=pltpu.MemorySpace.HBM),
        ],
        out_specs=pl.BlockSpec(memory_space=pltpu.VMEM),
        out_shape=jax.ShapeDtypeStruct((NCODE, NCODE), jnp.float32),
        scratch_shapes=[
            pltpu.VMEM((4, NBITS, 1, 1, CHUNK), jnp.float32),
            pltpu.VMEM((4, NBITS, 1, 1, CHUNK), jnp.float32),
            pltpu.SemaphoreType.DMA((2, 4)),
        ],
    )
    xp = jnp.transpose(y_pred, (1, 2, 0)).reshape(NBITS, nsteps, 1, CHUNK)
    xt = jnp.transpose(y_true, (1, 2, 0)).reshape(NBITS, nsteps, 1, CHUNK)
    xp = pltpu.with_memory_space_constraint(xp, pltpu.MemorySpace.HBM)
    xt = pltpu.with_memory_space_constraint(xt, pltpu.MemorySpace.HBM)
    return hist_call(xp, xt)


# speedup vs baseline: 1.5669x; 1.5669x over previous
import functools

import jax
import jax.numpy as jnp
from jax import lax
from jax.experimental import pallas as pl

NBITS = 4
NCODE = 16
CHUNK = 65536


def _hist_body(inv_total, nsteps, xp_ref, xt_ref, o_ref):
    g = pl.program_id(0)

    @pl.when(g == 0)
    def _init():
        o_ref[...] = jnp.zeros_like(o_ref)

    def encode(ref):
        acc = None
        for n in range(NBITS):
            bit = ref[n, 0]
            acc = bit if acc is None else acc * 2.0 + bit
        return acc

    cp = encode(xp_ref)
    ct = encode(xt_ref)
    io = lax.broadcasted_iota(jnp.int32, (NCODE, 1), 0).astype(jnp.float32)
    mp = (cp == io).astype(jnp.float32)
    mt = (ct == io).astype(jnp.float32)
    o_ref[...] += lax.dot_general(mp, mt, (((1,), (1,)), ((), ())),
                                  preferred_element_type=jnp.float32)

    @pl.when(g == nsteps - 1)
    def _norm():
        o_ref[...] = o_ref[...] * inv_total


@jax.jit
def kernel(y_pred, y_true):
    b = y_pred.shape[0]
    nsteps = b // CHUNK

    hist_call = pl.pallas_call(
        functools.partial(_hist_body, 1.0 / b, nsteps),
        grid=(nsteps,),
        in_specs=[
            pl.BlockSpec((NBITS, 1, 1, CHUNK), lambda g: (0, g, 0, 0)),
            pl.BlockSpec((NBITS, 1, 1, CHUNK), lambda g: (0, g, 0, 0)),
        ],
        out_specs=pl.BlockSpec((NCODE, NCODE), lambda g: (0, 0)),
        out_shape=jax.ShapeDtypeStruct((NCODE, NCODE), jnp.float32),
    )
    xp = jnp.transpose(y_pred, (1, 2, 0)).reshape(NBITS, nsteps, 1, CHUNK)
    xt = jnp.transpose(y_true, (1, 2, 0)).reshape(NBITS, nsteps, 1, CHUNK)
    return hist_call(xp, xt)


# speedup vs baseline: 1.8933x; 1.2083x over previous
import functools

import jax
import jax.numpy as jnp
from jax import lax
from jax.experimental import pallas as pl

NBITS = 4
NCODE = 16
CHUNK = 131072


def _hist_body(inv_total, nsteps, xp_ref, xt_ref, o_ref):
    g = pl.program_id(0)

    @pl.when(g == 0)
    def _init():
        o_ref[...] = jnp.zeros_like(o_ref)

    def encode(ref):
        acc = None
        for n in range(NBITS):
            bit = ref[n, 0]
            acc = bit if acc is None else acc * 2.0 + bit
        return acc

    cp = encode(xp_ref)
    ct = encode(xt_ref)
    io = lax.broadcasted_iota(jnp.int32, (NCODE, 1), 0).astype(jnp.float32)
    mp = (cp == io).astype(jnp.float32)
    mt = (ct == io).astype(jnp.float32)
    o_ref[...] += lax.dot_general(mp, mt, (((1,), (1,)), ((), ())),
                                  preferred_element_type=jnp.float32)

    @pl.when(g == nsteps - 1)
    def _norm():
        o_ref[...] = o_ref[...] * inv_total


@jax.jit
def kernel(y_pred, y_true):
    b = y_pred.shape[0]
    nsteps = b // CHUNK

    hist_call = pl.pallas_call(
        functools.partial(_hist_body, 1.0 / b, nsteps),
        grid=(nsteps,),
        in_specs=[
            pl.BlockSpec((NBITS, 1, 1, CHUNK), lambda g: (0, g, 0, 0)),
            pl.BlockSpec((NBITS, 1, 1, CHUNK), lambda g: (0, g, 0, 0)),
        ],
        out_specs=pl.BlockSpec((NCODE, NCODE), lambda g: (0, 0)),
        out_shape=jax.ShapeDtypeStruct((NCODE, NCODE), jnp.float32),
    )
    xp = jnp.transpose(y_pred, (1, 2, 0)).reshape(NBITS, nsteps, 1, CHUNK)
    xt = jnp.transpose(y_true, (1, 2, 0)).reshape(NBITS, nsteps, 1, CHUNK)
    return hist_call(xp, xt)


# speedup vs baseline: 1.9962x; 1.0543x over previous
import functools

import jax
import jax.numpy as jnp
from jax import lax
from jax.experimental import pallas as pl

NBITS = 4
NCODE = 16
CHUNK = 262144


def _hist_body(inv_total, nsteps, xp_ref, xt_ref, o_ref):
    g = pl.program_id(0)

    @pl.when(g == 0)
    def _init():
        o_ref[...] = jnp.zeros_like(o_ref)

    def encode(ref):
        acc = None
        for n in range(NBITS):
            bit = ref[n, 0]
            acc = bit if acc is None else acc * 2.0 + bit
        return acc

    cp = encode(xp_ref)
    ct = encode(xt_ref)
    io = lax.broadcasted_iota(jnp.int32, (NCODE, 1), 0).astype(jnp.float32)
    mp = (cp == io).astype(jnp.float32)
    mt = (ct == io).astype(jnp.float32)
    o_ref[...] += lax.dot_general(mp, mt, (((1,), (1,)), ((), ())),
                                  preferred_element_type=jnp.float32)

    @pl.when(g == nsteps - 1)
    def _norm():
        o_ref[...] = o_ref[...] * inv_total


@jax.jit
def kernel(y_pred, y_true):
    b = y_pred.shape[0]
    nsteps = b // CHUNK

    hist_call = pl.pallas_call(
        functools.partial(_hist_body, 1.0 / b, nsteps),
        grid=(nsteps,),
        in_specs=[
            pl.BlockSpec((NBITS, 1, 1, CHUNK), lambda g: (0, g, 0, 0)),
            pl.BlockSpec((NBITS, 1, 1, CHUNK), lambda g: (0, g, 0, 0)),
        ],
        out_specs=pl.BlockSpec((NCODE, NCODE), lambda g: (0, 0)),
        out_shape=jax.ShapeDtypeStruct((NCODE, NCODE), jnp.float32),
    )
    xp = jnp.transpose(y_pred, (1, 2, 0)).reshape(NBITS, nsteps, 1, CHUNK)
    xt = jnp.transpose(y_true, (1, 2, 0)).reshape(NBITS, nsteps, 1, CHUNK)
    return hist_call(xp, xt)
